# Initial kernel scaffold; baseline (speedup 1.0000x reference)
import jax
import jax.numpy as jnp
from jax.experimental import pallas as pl


def kernel(batch_x, batch_x_mark, support, enc0_Wg, enc0_bg, enc0_Wc, enc0_bc, enc1_Wg, enc1_bg, enc1_Wc, enc1_bc, dec0_Wg, dec0_bg, dec0_Wc, dec0_bc, dec1_Wg, dec1_bg, dec1_Wc, dec1_bc, proj_W, proj_b):
    raise NotImplementedError("write your pallas kernel here")


# speedup vs baseline: 14.3024x; 14.3024x over previous
import jax
import jax.numpy as jnp
from jax.experimental import pallas as pl
from jax.experimental.pallas import tpu as pltpu

N = 207
NP = 208
B = 32
H = 64
T = 12
BN = B * NP
DIN = 2
F32 = jnp.float32


def _dot(a, b):
    return jnp.dot(a, b, preferred_element_type=F32)


def _dcgru_body(x_ref, sup_ref,
                e0xg, e0hg, e0bg, e0xc, e0hc, e0bc,
                e1xg, e1hg, e1bg, e1xc, e1hc, e1bc,
                d0xg, d0hg, d0bg, d0xc, d0hc, d0bc,
                d1xg, d1hg, d1bg, d1xc, d1hc, d1bc,
                pw, pb,
                out_ref,
                h0, h1, dinp, rh_s, u_s, xf, hf, tmat):
    s = sup_ref[...]
    s2 = 2.0 * _dot(s, s) - jnp.eye(NP, dtype=F32)
    tmat[...] = jnp.concatenate([s, s2], axis=0)

    h0[...] = jnp.zeros_like(h0)
    h1[...] = jnp.zeros_like(h1)

    def cell(x_get, d, h_ref, wxg, whg, bg, wxc, whc, bc):
        tm = tmat[...]
        for b in range(B):
            xb = x_get(b)
            hb = h_ref[b]
            zb = jnp.concatenate([xb, hb], axis=1)
            dz = _dot(tm, zb)
            xf[b, :, 0:d] = xb
            xf[b, :, d:2 * d] = dz[0:NP, 0:d]
            xf[b, :, 2 * d:3 * d] = dz[NP:, 0:d]
            hf[b, :, 0:H] = hb
            hf[b, :, H:2 * H] = dz[0:NP, d:]
            hf[b, :, 2 * H:] = dz[NP:, d:]

        xfv = xf[:, :, 0:3 * d].reshape(BN, 3 * d)
        hfv = hf[...].reshape(BN, 3 * H)
        g = jax.nn.sigmoid(_dot(xfv, wxg[...]) + _dot(hfv, whg[...]) + bg[...])
        hv = h_ref[...].reshape(BN, H)
        rh_s[...] = (g[:, 0:H] * hv).reshape(B, NP, H)
        u_s[...] = g[:, H:].reshape(B, NP, H)

        for b in range(B):
            rhb = rh_s[b]
            dz = _dot(tm, rhb)
            hf[b, :, 0:H] = rhb
            hf[b, :, H:2 * H] = dz[0:NP]
            hf[b, :, 2 * H:] = dz[NP:]

        hfv2 = hf[...].reshape(BN, 3 * H)
        c = jnp.tanh(_dot(xfv, wxc[...]) + _dot(hfv2, whc[...]) + bc[...])
        u = u_s[...].reshape(BN, H)
        h_ref[...] = (u * hv + (1.0 - u) * c).reshape(B, NP, H)

    def enc_step(t, carry):
        cell(lambda b: x_ref[t, :, b * DIN:(b + 1) * DIN], DIN,
             h0, e0xg, e0hg, e0bg, e0xc, e0hc, e0bc)
        cell(lambda b: h0[b], H, h1, e1xg, e1hg, e1bg, e1xc, e1hc, e1bc)
        return carry
    jax.lax.fori_loop(0, T, enc_step, 0)

    dinp[...] = jnp.zeros_like(dinp)

    def dec_step(t, carry):
        cell(lambda b: dinp[:, b:b + 1], 1,
             h0, d0xg, d0hg, d0bg, d0xc, d0hc, d0bc)
        cell(lambda b: h0[b], H, h1, d1xg, d1hg, d1bg, d1xc, d1hc, d1bc)
        proj = _dot(h1[...].reshape(BN, H), pw[...]) + pb[...]
        proj = proj.reshape(B, NP, 1)
        for b in range(B):
            pb_col = proj[b]
            out_ref[t, :, b:b + 1] = pb_col
            dinp[:, b:b + 1] = pb_col
        return carry
    jax.lax.fori_loop(0, T, dec_step, 0)


def _split_w(w, d):
    w3 = w.reshape(d + H, 3, -1)
    wx = w3[:d].transpose(1, 0, 2).reshape(3 * d, -1)
    wh = w3[d:].transpose(1, 0, 2).reshape(3 * H, -1)
    return wx, wh


def kernel(batch_x, batch_x_mark, support,
           enc0_Wg, enc0_bg, enc0_Wc, enc0_bc,
           enc1_Wg, enc1_bg, enc1_Wc, enc1_bc,
           dec0_Wg, dec0_bg, dec0_Wc, dec0_bc,
           dec1_Wg, dec1_bg, dec1_Wc, dec1_bc,
           proj_W, proj_b):
    x = batch_x.transpose(1, 2, 0, 3).reshape(T, N, B * DIN)
    x = jnp.pad(x, ((0, 0), (0, NP - N), (0, 0)))
    sup = jnp.pad(support, ((0, NP - N), (0, NP - N)))

    e0xg, e0hg = _split_w(enc0_Wg, DIN)
    e0xc, e0hc = _split_w(enc0_Wc, DIN)
    e1xg, e1hg = _split_w(enc1_Wg, H)
    e1xc, e1hc = _split_w(enc1_Wc, H)
    d0xg, d0hg = _split_w(dec0_Wg, 1)
    d0xc, d0hc = _split_w(dec0_Wc, 1)
    d1xg, d1hg = _split_w(dec1_Wg, H)
    d1xc, d1hc = _split_w(dec1_Wc, H)

    args = (x, sup,
            e0xg, e0hg, enc0_bg.reshape(1, -1), e0xc, e0hc, enc0_bc.reshape(1, -1),
            e1xg, e1hg, enc1_bg.reshape(1, -1), e1xc, e1hc, enc1_bc.reshape(1, -1),
            d0xg, d0hg, dec0_bg.reshape(1, -1), d0xc, d0hc, dec0_bc.reshape(1, -1),
            d1xg, d1hg, dec1_bg.reshape(1, -1), d1xc, d1hc, dec1_bc.reshape(1, -1),
            proj_W, proj_b.reshape(1, 1))

    out = pl.pallas_call(
        _dcgru_body,
        out_shape=jax.ShapeDtypeStruct((T, NP, B), F32),
        scratch_shapes=[
            pltpu.VMEM((B, NP, H), F32),
            pltpu.VMEM((B, NP, H), F32),
            pltpu.VMEM((NP, B), F32),
            pltpu.VMEM((B, NP, H), F32),
            pltpu.VMEM((B, NP, H), F32),
            pltpu.VMEM((B, NP, 3 * H), F32),
            pltpu.VMEM((B, NP, 3 * H), F32),
            pltpu.VMEM((2 * NP, NP), F32),
        ],
    )(*args)

    return out[:, :N, :].transpose(2, 0, 1)[..., None]


# speedup vs baseline: 17.2964x; 1.2093x over previous
import jax
import jax.numpy as jnp
from jax.experimental import pallas as pl
from jax.experimental.pallas import tpu as pltpu

N = 207
NP = 208
B = 32
H = 64
T = 12
BN = B * NP
DIN = 2
F32 = jnp.float32


def _dot(a, b):
    return jnp.dot(a, b, preferred_element_type=F32)


def _dcgru_body(x_ref, sup_ref,
                e0x, e0hg, e0bg, e0hc, e0bc,
                e1x, e1hg, e1bg, e1hc, e1bc,
                d0x, d0hg, d0bg, d0hc, d0bc,
                d1x, d1hg, d1bg, d1hc, d1bc,
                pw, pb,
                out_ref,
                h0, h1, dinp, rh_s, u_s, xf, hf, tmat):
    s = sup_ref[...]
    s2 = 2.0 * _dot(s, s) - jnp.eye(NP, dtype=F32)
    tmat[...] = jnp.concatenate([s, s2], axis=0)

    h0[...] = jnp.zeros_like(h0)
    h1[...] = jnp.zeros_like(h1)

    def diffuse_state(tm, src_get, dst_ref):
        for b in range(0, B, 2):
            va = src_get(b)
            vb = src_get(b + 1)
            dz = _dot(tm, jnp.concatenate([va, vb], axis=1))
            dst_ref[b, :, 0:H] = va
            dst_ref[b, :, H:2 * H] = dz[0:NP, 0:H]
            dst_ref[b, :, 2 * H:] = dz[NP:, 0:H]
            dst_ref[b + 1, :, 0:H] = vb
            dst_ref[b + 1, :, H:2 * H] = dz[0:NP, H:]
            dst_ref[b + 1, :, 2 * H:] = dz[NP:, H:]

    def diffuse_x_all(tm, xall, d):
        dz = _dot(tm, xall)
        for b in range(B):
            sl = slice(b * d, (b + 1) * d)
            xf[b, :, 0:d] = xall[:, sl]
            xf[b, :, d:2 * d] = dz[0:NP, sl]
            xf[b, :, 2 * d:3 * d] = dz[NP:, sl]

    def cell(d, h_ref, fill_x, wx, whg, bg, whc, bc):
        tm = tmat[...]
        fill_x(tm)
        diffuse_state(tm, lambda b: h_ref[b], hf)

        xfv = xf[:, :, 0:3 * d].reshape(BN, 3 * d)
        hfv = hf[...].reshape(BN, 3 * H)
        xc = _dot(xfv, wx[...])
        g = jax.nn.sigmoid(xc[:, 0:2 * H] + _dot(hfv, whg[...]) + bg[...])
        hv = h_ref[...].reshape(BN, H)
        rh_s[...] = (g[:, 0:H] * hv).reshape(B, NP, H)
        u_s[...] = g[:, H:].reshape(B, NP, H)

        diffuse_state(tm, lambda b: rh_s[b], hf)

        hfv2 = hf[...].reshape(BN, 3 * H)
        c = jnp.tanh(xc[:, 2 * H:] + _dot(hfv2, whc[...]) + bc[...])
        u = u_s[...].reshape(BN, H)
        h_ref[...] = (u * hv + (1.0 - u) * c).reshape(B, NP, H)

    def enc_step(t, carry):
        cell(DIN, h0, lambda tm: diffuse_x_all(tm, x_ref[t], DIN),
             e0x, e0hg, e0bg, e0hc, e0bc)
        cell(H, h1, lambda tm: diffuse_state(tm, lambda b: h0[b], xf),
             e1x, e1hg, e1bg, e1hc, e1bc)
        return carry
    jax.lax.fori_loop(0, T, enc_step, 0)

    dinp[...] = jnp.zeros_like(dinp)

    def dec_step(t, carry):
        cell(1, h0, lambda tm: diffuse_x_all(tm, dinp[...], 1),
             d0x, d0hg, d0bg, d0hc, d0bc)
        cell(H, h1, lambda tm: diffuse_state(tm, lambda b: h0[b], xf),
             d1x, d1hg, d1bg, d1hc, d1bc)
        proj = _dot(h1[...].reshape(BN, H), pw[...]) + pb[...]
        proj = proj.reshape(B, NP, 1)
        for b in range(B):
            pcol = proj[b]
            out_ref[t, :, b:b + 1] = pcol
            dinp[:, b:b + 1] = pcol
        return carry
    jax.lax.fori_loop(0, T, dec_step, 0)


def _split_w(wg, wc, d):
    g3 = wg.reshape(d + H, 3, -1)
    c3 = wc.reshape(d + H, 3, -1)
    wxg = g3[:d].transpose(1, 0, 2).reshape(3 * d, -1)
    wxc = c3[:d].transpose(1, 0, 2).reshape(3 * d, -1)
    wx = jnp.concatenate([wxg, wxc], axis=1)
    whg = g3[d:].transpose(1, 0, 2).reshape(3 * H, -1)
    whc = c3[d:].transpose(1, 0, 2).reshape(3 * H, -1)
    return wx, whg, whc


def kernel(batch_x, batch_x_mark, support,
           enc0_Wg, enc0_bg, enc0_Wc, enc0_bc,
           enc1_Wg, enc1_bg, enc1_Wc, enc1_bc,
           dec0_Wg, dec0_bg, dec0_Wc, dec0_bc,
           dec1_Wg, dec1_bg, dec1_Wc, dec1_bc,
           proj_W, proj_b):
    x = batch_x.transpose(1, 2, 0, 3).reshape(T, N, B * DIN)
    x = jnp.pad(x, ((0, 0), (0, NP - N), (0, 0)))
    sup = jnp.pad(support, ((0, NP - N), (0, NP - N)))

    e0x, e0hg, e0hc = _split_w(enc0_Wg, enc0_Wc, DIN)
    e1x, e1hg, e1hc = _split_w(enc1_Wg, enc1_Wc, H)
    d0x, d0hg, d0hc = _split_w(dec0_Wg, dec0_Wc, 1)
    d1x, d1hg, d1hc = _split_w(dec1_Wg, dec1_Wc, H)

    args = (x, sup,
            e0x, e0hg, enc0_bg.reshape(1, -1), e0hc, enc0_bc.reshape(1, -1),
            e1x, e1hg, enc1_bg.reshape(1, -1), e1hc, enc1_bc.reshape(1, -1),
            d0x, d0hg, dec0_bg.reshape(1, -1), d0hc, dec0_bc.reshape(1, -1),
            d1x, d1hg, dec1_bg.reshape(1, -1), d1hc, dec1_bc.reshape(1, -1),
            proj_W, proj_b.reshape(1, 1))

    out = pl.pallas_call(
        _dcgru_body,
        out_shape=jax.ShapeDtypeStruct((T, NP, B), F32),
        scratch_shapes=[
            pltpu.VMEM((B, NP, H), F32),
            pltpu.VMEM((B, NP, H), F32),
            pltpu.VMEM((NP, B), F32),
            pltpu.VMEM((B, NP, H), F32),
            pltpu.VMEM((B, NP, H), F32),
            pltpu.VMEM((B, NP, 3 * H), F32),
            pltpu.VMEM((B, NP, 3 * H), F32),
            pltpu.VMEM((2 * NP, NP), F32),
        ],
    )(*args)

    return out[:, :N, :].transpose(2, 0, 1)[..., None]


# speedup vs baseline: 17.3080x; 1.0007x over previous
import jax
import jax.numpy as jnp
from jax.experimental import pallas as pl
from jax.experimental.pallas import tpu as pltpu

N = 207
NP = 208
B = 32
H = 64
T = 12
BN = B * NP
DIN = 2
F32 = jnp.float32
BF16 = jnp.bfloat16


def _dot(a, b):
    return jnp.dot(a, b, preferred_element_type=F32)


def _dcgru_body(x_ref, sup_ref,
                e0x, e0hg, e0bg, e0hc, e0bc,
                e1x, e1hg, e1bg, e1hc, e1bc,
                d0x, d0hg, d0bg, d0hc, d0bc,
                d1x, d1hg, d1bg, d1hc, d1bc,
                pw, pb,
                out_ref,
                h0, h1, dinp, rh_s, u_s, xf, hf, tmat):
    s = sup_ref[...]
    s2 = 2.0 * _dot(s, s) - jnp.eye(NP, dtype=F32)
    tmat[...] = jnp.concatenate([s, s2], axis=0).astype(BF16)

    h0[...] = jnp.zeros_like(h0)
    h1[...] = jnp.zeros_like(h1)

    def diffuse_state(tm, src_get, dst_ref):
        for b in range(0, B, 2):
            va = src_get(b)
            vb = src_get(b + 1)
            dz = _dot(tm, jnp.concatenate([va, vb], axis=1).astype(BF16))
            dst_ref[b, :, 0:H] = va
            dst_ref[b, :, H:2 * H] = dz[0:NP, 0:H]
            dst_ref[b, :, 2 * H:] = dz[NP:, 0:H]
            dst_ref[b + 1, :, 0:H] = vb
            dst_ref[b + 1, :, H:2 * H] = dz[0:NP, H:]
            dst_ref[b + 1, :, 2 * H:] = dz[NP:, H:]

    def diffuse_x_all(tm, xall, d):
        dz = _dot(tm, xall.astype(BF16))
        for b in range(B):
            sl = slice(b * d, (b + 1) * d)
            xf[b, :, 0:d] = xall[:, sl]
            xf[b, :, d:2 * d] = dz[0:NP, sl]
            xf[b, :, 2 * d:3 * d] = dz[NP:, sl]

    def cell(d, h_ref, fill_x, wx, whg, bg, whc, bc):
        tm = tmat[...]
        fill_x(tm)
        diffuse_state(tm, lambda b: h_ref[b], hf)

        xfv = xf[:, :, 0:3 * d].reshape(BN, 3 * d)
        hfv = hf[...].reshape(BN, 3 * H)
        xc = _dot(xfv, wx[...])
        g = jax.nn.sigmoid(xc[:, 0:2 * H] + _dot(hfv, whg[...]) + bg[...])
        hv = h_ref[...].reshape(BN, H)
        rh_s[...] = (g[:, 0:H] * hv).reshape(B, NP, H)
        u_s[...] = g[:, H:].reshape(B, NP, H)

        diffuse_state(tm, lambda b: rh_s[b], hf)

        hfv2 = hf[...].reshape(BN, 3 * H)
        c = jnp.tanh(xc[:, 2 * H:] + _dot(hfv2, whc[...]) + bc[...])
        u = u_s[...].reshape(BN, H)
        h_ref[...] = (u * hv + (1.0 - u) * c).reshape(B, NP, H)

    def enc_step(t, carry):
        cell(DIN, h0, lambda tm: diffuse_x_all(tm, x_ref[t], DIN),
             e0x, e0hg, e0bg, e0hc, e0bc)
        cell(H, h1, lambda tm: diffuse_state(tm, lambda b: h0[b], xf),
             e1x, e1hg, e1bg, e1hc, e1bc)
        return carry
    jax.lax.fori_loop(0, T, enc_step, 0)

    dinp[...] = jnp.zeros_like(dinp)

    def dec_step(t, carry):
        cell(1, h0, lambda tm: diffuse_x_all(tm, dinp[...], 1),
             d0x, d0hg, d0bg, d0hc, d0bc)
        cell(H, h1, lambda tm: diffuse_state(tm, lambda b: h0[b], xf),
             d1x, d1hg, d1bg, d1hc, d1bc)
        proj = _dot(h1[...].reshape(BN, H), pw[...]) + pb[...]
        proj = proj.reshape(B, NP, 1)
        for b in range(B):
            pcol = proj[b]
            out_ref[t, :, b:b + 1] = pcol
            dinp[:, b:b + 1] = pcol
        return carry
    jax.lax.fori_loop(0, T, dec_step, 0)


def _split_w(wg, wc, d):
    g3 = wg.reshape(d + H, 3, -1)
    c3 = wc.reshape(d + H, 3, -1)
    wxg = g3[:d].transpose(1, 0, 2).reshape(3 * d, -1)
    wxc = c3[:d].transpose(1, 0, 2).reshape(3 * d, -1)
    wx = jnp.concatenate([wxg, wxc], axis=1)
    whg = g3[d:].transpose(1, 0, 2).reshape(3 * H, -1)
    whc = c3[d:].transpose(1, 0, 2).reshape(3 * H, -1)
    return wx, whg, whc


def kernel(batch_x, batch_x_mark, support,
           enc0_Wg, enc0_bg, enc0_Wc, enc0_bc,
           enc1_Wg, enc1_bg, enc1_Wc, enc1_bc,
           dec0_Wg, dec0_bg, dec0_Wc, dec0_bc,
           dec1_Wg, dec1_bg, dec1_Wc, dec1_bc,
           proj_W, proj_b):
    x = batch_x.transpose(1, 2, 0, 3).reshape(T, N, B * DIN)
    x = jnp.pad(x, ((0, 0), (0, NP - N), (0, 0)))
    sup = jnp.pad(support, ((0, NP - N), (0, NP - N)))

    e0x, e0hg, e0hc = _split_w(enc0_Wg, enc0_Wc, DIN)
    e1x, e1hg, e1hc = _split_w(enc1_Wg, enc1_Wc, H)
    d0x, d0hg, d0hc = _split_w(dec0_Wg, dec0_Wc, 1)
    d1x, d1hg, d1hc = _split_w(dec1_Wg, dec1_Wc, H)

    args = (x, sup,
            e0x, e0hg, enc0_bg.reshape(1, -1), e0hc, enc0_bc.reshape(1, -1),
            e1x, e1hg, enc1_bg.reshape(1, -1), e1hc, enc1_bc.reshape(1, -1),
            d0x, d0hg, dec0_bg.reshape(1, -1), d0hc, dec0_bc.reshape(1, -1),
            d1x, d1hg, dec1_bg.reshape(1, -1), d1hc, dec1_bc.reshape(1, -1),
            proj_W, proj_b.reshape(1, 1))

    out = pl.pallas_call(
        _dcgru_body,
        out_shape=jax.ShapeDtypeStruct((T, NP, B), F32),
        scratch_shapes=[
            pltpu.VMEM((B, NP, H), F32),
            pltpu.VMEM((B, NP, H), F32),
            pltpu.VMEM((NP, B), F32),
            pltpu.VMEM((B, NP, H), F32),
            pltpu.VMEM((B, NP, H), F32),
            pltpu.VMEM((B, NP, 3 * H), F32),
            pltpu.VMEM((B, NP, 3 * H), F32),
            pltpu.VMEM((2 * NP, NP), BF16),
        ],
    )(*args)

    return out[:, :N, :].transpose(2, 0, 1)[..., None]


# speedup vs baseline: 18.2478x; 1.0543x over previous
import jax
import jax.numpy as jnp
from jax.experimental import pallas as pl
from jax.experimental.pallas import tpu as pltpu

N = 207
NP = 208
B = 32
H = 64
T = 12
BN = B * NP
DIN = 2
FW = 3 * H
F32 = jnp.float32
BF16 = jnp.bfloat16


def _dot(a, b):
    return jnp.dot(a, b, preferred_element_type=F32)


def _dcgru_body(x_ref, sup_ref,
                e0g, e0bg, e0c, e0bc,
                e1g, e1bg, e1c, e1bc,
                d0g, d0bg, d0c, d0bc,
                d1g, d1bg, d1c, d1bc,
                pw, pb,
                out_ref,
                h0, h1, dinp, rh_s, u_s, zf, tmat):
    s = sup_ref[...]
    s2 = 2.0 * _dot(s, s) - jnp.eye(NP, dtype=F32)
    tmat[...] = jnp.concatenate([s, s2], axis=0).astype(BF16)

    h0[...] = jnp.zeros_like(h0)
    h1[...] = jnp.zeros_like(h1)

    def diffuse_state(tm, src_get, off):
        for b in range(0, B, 2):
            va = src_get(b)
            vb = src_get(b + 1)
            dz = _dot(tm, jnp.concatenate([va, vb], axis=1).astype(BF16))
            zf[b, :, off:off + H] = va
            zf[b, :, off + H:off + 2 * H] = dz[0:NP, 0:H]
            zf[b, :, off + 2 * H:off + 3 * H] = dz[NP:, 0:H]
            zf[b + 1, :, off:off + H] = vb
            zf[b + 1, :, off + H:off + 2 * H] = dz[0:NP, H:]
            zf[b + 1, :, off + 2 * H:off + 3 * H] = dz[NP:, H:]

    def diffuse_x_all(tm, xall, d):
        dz = _dot(tm, xall.astype(BF16))
        for b in range(B):
            sl = slice(b * d, (b + 1) * d)
            zf[b, :, FW:FW + d] = xall[:, sl]
            zf[b, :, FW + d:FW + 2 * d] = dz[0:NP, sl]
            zf[b, :, FW + 2 * d:FW + 3 * d] = dz[NP:, sl]

    def cell(d, h_ref, fill_x, wg, bg, wc, bc):
        tm = tmat[...]
        fill_x(tm)
        diffuse_state(tm, lambda b: h_ref[b], 0)

        w = FW + 3 * d
        zv = zf[:, :, 0:w].reshape(BN, w)
        g = jax.nn.sigmoid(_dot(zv.astype(BF16), wg[...]) + bg[...])
        hv = h_ref[...].reshape(BN, H)
        rh_s[...] = (g[:, 0:H] * hv).reshape(B, NP, H)
        u_s[...] = g[:, H:].reshape(B, NP, H)

        diffuse_state(tm, lambda b: rh_s[b], 0)

        zv2 = zf[:, :, 0:w].reshape(BN, w)
        c = jnp.tanh(_dot(zv2, wc[...]) + bc[...])
        u = u_s[...].reshape(BN, H)
        h_ref[...] = (c + u * (hv - c)).reshape(B, NP, H)

    def enc_step(t, carry):
        cell(DIN, h0, lambda tm: diffuse_x_all(tm, x_ref[t], DIN),
             e0g, e0bg, e0c, e0bc)
        cell(H, h1, lambda tm: diffuse_state(tm, lambda b: h0[b], FW),
             e1g, e1bg, e1c, e1bc)
        return carry
    jax.lax.fori_loop(0, T, enc_step, 0)

    dinp[...] = jnp.zeros_like(dinp)

    def dec_step(t, carry):
        cell(1, h0, lambda tm: diffuse_x_all(tm, dinp[...], 1),
             d0g, d0bg, d0c, d0bc)
        cell(H, h1, lambda tm: diffuse_state(tm, lambda b: h0[b], FW),
             d1g, d1bg, d1c, d1bc)
        proj = _dot(h1[...].reshape(BN, H), pw[...]) + pb[...]
        proj = proj.reshape(B, NP, 1)
        for b in range(B):
            pcol = proj[b]
            out_ref[t, :, b:b + 1] = pcol
            dinp[:, b:b + 1] = pcol
        return carry
    jax.lax.fori_loop(0, T, dec_step, 0)


def _pack_w(w, d, dtype):
    w3 = w.reshape(d + H, 3, -1)
    wh = w3[d:].transpose(1, 0, 2).reshape(3 * H, -1)
    wx = w3[:d].transpose(1, 0, 2).reshape(3 * d, -1)
    return jnp.concatenate([wh, wx], axis=0).astype(dtype)


def kernel(batch_x, batch_x_mark, support,
           enc0_Wg, enc0_bg, enc0_Wc, enc0_bc,
           enc1_Wg, enc1_bg, enc1_Wc, enc1_bc,
           dec0_Wg, dec0_bg, dec0_Wc, dec0_bc,
           dec1_Wg, dec1_bg, dec1_Wc, dec1_bc,
           proj_W, proj_b):
    x = batch_x.transpose(1, 2, 0, 3).reshape(T, N, B * DIN)
    x = jnp.pad(x, ((0, 0), (0, NP - N), (0, 0)))
    sup = jnp.pad(support, ((0, NP - N), (0, NP - N)))

    args = (x, sup,
            _pack_w(enc0_Wg, DIN, BF16), enc0_bg.reshape(1, -1),
            _pack_w(enc0_Wc, DIN, F32), enc0_bc.reshape(1, -1),
            _pack_w(enc1_Wg, H, BF16), enc1_bg.reshape(1, -1),
            _pack_w(enc1_Wc, H, F32), enc1_bc.reshape(1, -1),
            _pack_w(dec0_Wg, 1, BF16), dec0_bg.reshape(1, -1),
            _pack_w(dec0_Wc, 1, F32), dec0_bc.reshape(1, -1),
            _pack_w(dec1_Wg, H, BF16), dec1_bg.reshape(1, -1),
            _pack_w(dec1_Wc, H, F32), dec1_bc.reshape(1, -1),
            proj_W, proj_b.reshape(1, 1))

    out = pl.pallas_call(
        _dcgru_body,
        out_shape=jax.ShapeDtypeStruct((T, NP, B), F32),
        scratch_shapes=[
            pltpu.VMEM((B, NP, H), F32),
            pltpu.VMEM((B, NP, H), F32),
            pltpu.VMEM((NP, B), F32),
            pltpu.VMEM((B, NP, H), F32),
            pltpu.VMEM((B, NP, H), F32),
            pltpu.VMEM((B, NP, 2 * FW), F32),
            pltpu.VMEM((2 * NP, NP), BF16),
        ],
    )(*args)

    return out[:, :N, :].transpose(2, 0, 1)[..., None]
